# Initial kernel scaffold; baseline (speedup 1.0000x reference)
import jax
import jax.numpy as jnp
from jax.experimental import pallas as pl


def kernel(x, edge_index, batch, root_mask, eps1, w1a, b1a, w1b, b1b, eps2, w2a, b2a, w2b, b2b, eps3, w3a, b3a, w3b, b3b):
    raise NotImplementedError("write your pallas kernel here")


# speedup vs baseline: 6.6757x; 6.6757x over previous
import functools

import jax
import jax.numpy as jnp
from jax import lax
from jax.experimental import pallas as pl
from jax.experimental.pallas import tpu as pltpu
from jax.experimental.pallas import tpu_sc as plsc

N = 10000
E = 320000
D = 128

NC = 2
NS = 16
NW = NC * NS
EPW = E // NW
CH = 80
NCHUNK = EPW // CH
RPT = 624
TAIL = N - NS * RPT
ZR = 16

_MESH = plsc.VectorSubcoreMesh(
    core_axis_name="c", subcore_axis_name="s", num_cores=NC, num_subcores=NS)


@functools.partial(
    pl.kernel,
    out_type=jax.ShapeDtypeStruct((NC * N, D), jnp.float32),
    mesh=_MESH,
    scratch_types=[
        pltpu.VMEM((NCHUNK, CH), jnp.int32),
        pltpu.VMEM((NCHUNK, CH), jnp.int32),
        pltpu.VMEM((CH, D), jnp.float32),
        pltpu.VMEM((ZR, D), jnp.float32),
        pltpu.SemaphoreType.DMA,
        pltpu.VMEM_SHARED((N, D), jnp.float32),
    ],
)
def _sc_agg(h_hbm, edges_hbm, zeros_hbm, out_hbm,
            src_v, dst_v, rows_v, zero_v, sem, agg_sh):
    c = lax.axis_index("c")
    s = lax.axis_index("s")
    pltpu.sync_copy(zeros_hbm, zero_v)

    def zbody(k, carry):
        pltpu.sync_copy(zero_v, agg_sh.at[pl.ds(s * RPT + k * ZR, ZR)])
        return carry

    lax.fori_loop(0, RPT // ZR, zbody, 0)

    @pl.when(s == NS - 1)
    def _():
        pltpu.sync_copy(zero_v, agg_sh.at[pl.ds(NS * RPT, TAIL)])

    pltpu.sync_copy(edges_hbm.at[0, s * NC + c], src_v)
    pltpu.sync_copy(edges_hbm.at[1, s * NC + c], dst_v)
    plsc.subcore_barrier()

    def body(ci, carry):
        pltpu.async_copy(h_hbm.at[src_v.at[ci]], rows_v, sem).wait()
        pltpu.sync_copy(rows_v, agg_sh.at[dst_v.at[ci]], add=True)
        return carry

    lax.fori_loop(0, NCHUNK, body, 0)
    plsc.subcore_barrier()
    pltpu.sync_copy(agg_sh.at[pl.ds(s * RPT, RPT)],
                    out_hbm.at[pl.ds(c * N + s * RPT, RPT)])

    @pl.when(s == NS - 1)
    def _():
        pltpu.sync_copy(agg_sh.at[pl.ds(NS * RPT, TAIL)],
                        out_hbm.at[pl.ds(c * N + NS * RPT, TAIL)])


BLK = 2000


def _mlp_body(last, eps_ref, h_ref, a0_ref, a1_ref,
              w1_ref, b1_ref, w2_ref, b2_ref, m_ref, o_ref):
    z = h_ref[...] * eps_ref[...] + a0_ref[...] + a1_ref[...]
    y = jnp.dot(z, w1_ref[...], preferred_element_type=jnp.float32)
    y = jnp.maximum(y + b1_ref[...], 0.0)
    o = jnp.dot(y, w2_ref[...], preferred_element_type=jnp.float32)
    o = o + b2_ref[...]
    if last:
        o = jnp.where(m_ref[...] > 0.0, o, 0.0)
    else:
        o = jnp.maximum(o, 0.0)
    o_ref[...] = o


def _mlp(parts, h, eps11, w1, b1r, w2, b2r, mask2d, last):
    body = functools.partial(_mlp_body, last)
    nb = N // BLK
    return pl.pallas_call(
        body,
        grid=(nb,),
        in_specs=[
            pl.BlockSpec((1, 1), lambda i: (0, 0)),
            pl.BlockSpec((BLK, D), lambda i: (i, 0)),
            pl.BlockSpec((BLK, D), lambda i: (i, 0)),
            pl.BlockSpec((BLK, D), lambda i: (i + nb, 0)),
            pl.BlockSpec((D, D), lambda i: (0, 0)),
            pl.BlockSpec((1, D), lambda i: (0, 0)),
            pl.BlockSpec((D, D), lambda i: (0, 0)),
            pl.BlockSpec((1, D), lambda i: (0, 0)),
            pl.BlockSpec((BLK, 1), lambda i: (i, 0)),
        ],
        out_specs=pl.BlockSpec((BLK, D), lambda i: (i, 0)),
        out_shape=jax.ShapeDtypeStruct((N, D), jnp.float32),
    )(eps11, h, parts, parts, w1, b1r, w2, b2r, mask2d)


def kernel(x, edge_index, batch, root_mask,
           eps1, w1a, b1a, w1b, b1b,
           eps2, w2a, b2a, w2b, b2b,
           eps3, w3a, b3a, w3b, b3b):
    er = edge_index.astype(jnp.int32).reshape(2, NW, NCHUNK, CH)
    zeros_blk = jnp.zeros((ZR, D), jnp.float32)
    mask2d = root_mask.astype(jnp.float32).reshape(N, 1)
    h = x
    layers = ((eps1, w1a, b1a, w1b, b1b, False),
              (eps2, w2a, b2a, w2b, b2b, False),
              (eps3, w3a, b3a, w3b, b3b, True))
    for eps, wa, ba, wb, bb, last in layers:
        parts = _sc_agg(h, er, zeros_blk)
        h = _mlp(parts, h,
                 (1.0 + eps).astype(jnp.float32).reshape(1, 1),
                 wa, ba.reshape(1, D), wb, bb.reshape(1, D), mask2d, last)
    return h


# speedup vs baseline: 11.6777x; 1.7493x over previous
import functools

import jax
import jax.numpy as jnp
from jax import lax
from jax.experimental import pallas as pl
from jax.experimental.pallas import tpu as pltpu
from jax.experimental.pallas import tpu_sc as plsc

N = 10000
E = 320000
D = 128

NC = 2
NS = 16
NW = NC * NS
EPW = E // NW
CH = 125
NCHUNK = EPW // CH
NPH = 2
CPP = NCHUNK // NPH
RPT = 624
TAIL = N - NS * RPT
ZR = 16

_MESH = plsc.VectorSubcoreMesh(
    core_axis_name="c", subcore_axis_name="s", num_cores=NC, num_subcores=NS)


@functools.partial(
    pl.kernel,
    out_type=jax.ShapeDtypeStruct((NC * N, D), jnp.float32),
    mesh=_MESH,
    scratch_types=[
        pltpu.VMEM((CPP, CH), jnp.int32),
        pltpu.VMEM((CPP, CH), jnp.int32),
        pltpu.VMEM((CH, D), jnp.float32),
        pltpu.VMEM((CH, D), jnp.float32),
        pltpu.VMEM((ZR, D), jnp.float32),
        pltpu.SemaphoreType.DMA,
        pltpu.SemaphoreType.DMA,
        pltpu.VMEM_SHARED((N, D), jnp.float32),
    ],
)
def _sc_agg(h_hbm, edges_hbm, zeros_hbm, out_hbm,
            src_v, dst_v, rows0_v, rows1_v, zero_v, sem0, sem1, agg_sh):
    c = lax.axis_index("c")
    s = lax.axis_index("s")
    pltpu.sync_copy(zeros_hbm, zero_v)

    def zbody(k, carry):
        pltpu.sync_copy(zero_v, agg_sh.at[pl.ds(s * RPT + k * ZR, ZR)])
        return carry

    lax.fori_loop(0, RPT // ZR, zbody, 0)

    @pl.when(s == NS - 1)
    def _():
        pltpu.sync_copy(zero_v, agg_sh.at[pl.ds(NS * RPT, TAIL)])

    plsc.subcore_barrier()
    wid = s * NC + c
    for ph in range(NPH):
        pltpu.sync_copy(edges_hbm.at[0, wid, ph], src_v)
        pltpu.sync_copy(edges_hbm.at[1, wid, ph], dst_v)

        pltpu.async_copy(h_hbm.at[src_v.at[0]], rows0_v, sem0)

        def body(i, carry):
            a = 2 * i
            b = a + 1
            pltpu.async_copy(h_hbm.at[src_v.at[b]], rows1_v, sem1)
            pltpu.make_async_copy(h_hbm.at[src_v.at[a]], rows0_v, sem0).wait()
            pltpu.sync_copy(rows0_v, agg_sh.at[dst_v.at[a]], add=True)

            @pl.when(i < CPP // 2 - 1)
            def _():
                pltpu.async_copy(h_hbm.at[src_v.at[a + 2]], rows0_v, sem0)

            pltpu.make_async_copy(h_hbm.at[src_v.at[b]], rows1_v, sem1).wait()
            pltpu.sync_copy(rows1_v, agg_sh.at[dst_v.at[b]], add=True)
            return carry

        lax.fori_loop(0, CPP // 2, body, 0)
    plsc.subcore_barrier()
    pltpu.sync_copy(agg_sh.at[pl.ds(s * RPT, RPT)],
                    out_hbm.at[pl.ds(c * N + s * RPT, RPT)])

    @pl.when(s == NS - 1)
    def _():
        pltpu.sync_copy(agg_sh.at[pl.ds(NS * RPT, TAIL)],
                        out_hbm.at[pl.ds(c * N + NS * RPT, TAIL)])


BLK = 2000


def _mlp_body(last, eps_ref, h_ref, a0_ref, a1_ref,
              w1_ref, b1_ref, w2_ref, b2_ref, m_ref, o_ref):
    z = h_ref[...] * eps_ref[...] + a0_ref[...] + a1_ref[...]
    y = jnp.dot(z, w1_ref[...], preferred_element_type=jnp.float32)
    y = jnp.maximum(y + b1_ref[...], 0.0)
    o = jnp.dot(y, w2_ref[...], preferred_element_type=jnp.float32)
    o = o + b2_ref[...]
    if last:
        o = jnp.where(m_ref[...] > 0.0, o, 0.0)
    else:
        o = jnp.maximum(o, 0.0)
    o_ref[...] = o


def _mlp(parts, h, eps11, w1, b1r, w2, b2r, mask2d, last):
    body = functools.partial(_mlp_body, last)
    nb = N // BLK
    return pl.pallas_call(
        body,
        grid=(nb,),
        in_specs=[
            pl.BlockSpec((1, 1), lambda i: (0, 0)),
            pl.BlockSpec((BLK, D), lambda i: (i, 0)),
            pl.BlockSpec((BLK, D), lambda i: (i, 0)),
            pl.BlockSpec((BLK, D), lambda i: (i + nb, 0)),
            pl.BlockSpec((D, D), lambda i: (0, 0)),
            pl.BlockSpec((1, D), lambda i: (0, 0)),
            pl.BlockSpec((D, D), lambda i: (0, 0)),
            pl.BlockSpec((1, D), lambda i: (0, 0)),
            pl.BlockSpec((BLK, 1), lambda i: (i, 0)),
        ],
        out_specs=pl.BlockSpec((BLK, D), lambda i: (i, 0)),
        out_shape=jax.ShapeDtypeStruct((N, D), jnp.float32),
    )(eps11, h, parts, parts, w1, b1r, w2, b2r, mask2d)


def kernel(x, edge_index, batch, root_mask,
           eps1, w1a, b1a, w1b, b1b,
           eps2, w2a, b2a, w2b, b2b,
           eps3, w3a, b3a, w3b, b3b):
    er = edge_index.astype(jnp.int32).reshape(2, NW, NPH, CPP, CH)
    zeros_blk = jnp.zeros((ZR, D), jnp.float32)
    mask2d = root_mask.astype(jnp.float32).reshape(N, 1)
    h = x
    layers = ((eps1, w1a, b1a, w1b, b1b, False),
              (eps2, w2a, b2a, w2b, b2b, False),
              (eps3, w3a, b3a, w3b, b3b, True))
    for eps, wa, ba, wb, bb, last in layers:
        parts = _sc_agg(h, er, zeros_blk)
        h = _mlp(parts, h,
                 (1.0 + eps).astype(jnp.float32).reshape(1, 1),
                 wa, ba.reshape(1, D), wb, bb.reshape(1, D), mask2d, last)
    return h
